# Initial kernel scaffold; baseline (speedup 1.0000x reference)
import jax
import jax.numpy as jnp
from jax.experimental import pallas as pl


def kernel(x, edge_index, W_in, b_in, Wl0, Wr0, b0, Wl1, Wr1, b1, Wl2, Wr2, b2):
    raise NotImplementedError("write your pallas kernel here")


# speedup vs baseline: 3.1918x; 3.1918x over previous
import functools

import jax
import jax.numpy as jnp
from jax import lax
from jax.experimental import pallas as pl
from jax.experimental.pallas import tpu as pltpu
from jax.experimental.pallas import tpu_sc as plsc

N = 10000
E = 320000
D_IN = 128
D_H = 128
D_OUT = 64

NPAD = 10240
EPAD = 327680
EROWS = EPAD // 128
NW = 32
EROWS_W = EROWS // NW
ROWS_TILE = NPAD // 16
DEG_W = 128

_mesh = plsc.VectorSubcoreMesh(core_axis_name="c", subcore_axis_name="s")


def _zero_fill(ref, nrows, ncols):
    z = jnp.zeros((16,), jnp.float32)
    kcols = ncols // 16

    def body(t, carry):
        i = t // kcols
        k = t % kcols
        ref[i, pl.ds(k * 16, 16)] = z
        return carry

    lax.fori_loop(0, nrows * kcols, body, 0)


def _sc_scatter_sum(m, srcs, dsts, dm):
    @functools.partial(
        pl.kernel,
        out_type=jax.ShapeDtypeStruct((2 * NPAD, dm), jnp.float32),
        mesh=_mesh,
        scratch_types=[
            pltpu.VMEM((EROWS_W, 128), jnp.int32),
            pltpu.VMEM((EROWS_W, 128), jnp.int32),
            pltpu.VMEM((128, dm), jnp.float32),
            pltpu.VMEM((64, dm), jnp.float32),
            pltpu.VMEM_SHARED((NPAD, dm), jnp.float32),
            pltpu.SemaphoreType.DMA,
        ],
    )
    def scat(m_hbm, srcs_hbm, dsts_hbm, out_hbm, sidx, didx, rows, zbuf, acc, sem):
        cid = lax.axis_index("c")
        sid = lax.axis_index("s")
        wid = cid * 16 + sid
        base = wid * EROWS_W

        pltpu.sync_copy(srcs_hbm.at[pl.ds(base, EROWS_W)], sidx)
        pltpu.sync_copy(dsts_hbm.at[pl.ds(base, EROWS_W)], didx)

        _zero_fill(zbuf, 64, dm)
        for t in range(ROWS_TILE // 64):
            pltpu.sync_copy(zbuf, acc.at[pl.ds(sid * ROWS_TILE + t * 64, 64)])
        plsc.subcore_barrier()

        def body(j, carry):
            pltpu.async_copy(m_hbm.at[sidx.at[j]], rows, sem).wait()
            pltpu.sync_copy(rows, acc.at[didx.at[j]], add=True)
            return carry

        lax.fori_loop(0, EROWS_W, body, 0)
        plsc.subcore_barrier()

        pltpu.sync_copy(
            acc.at[pl.ds(sid * ROWS_TILE, ROWS_TILE)],
            out_hbm.at[pl.ds(cid * NPAD + sid * ROWS_TILE, ROWS_TILE)],
        )

    return scat(m, srcs, dsts)


def _sc_degree(dsts):
    @functools.partial(
        pl.kernel,
        out_type=jax.ShapeDtypeStruct((2 * NPAD, DEG_W), jnp.float32),
        mesh=_mesh,
        scratch_types=[
            pltpu.VMEM((EROWS_W, 128), jnp.int32),
            pltpu.VMEM((128, DEG_W), jnp.float32),
            pltpu.VMEM((64, DEG_W), jnp.float32),
            pltpu.VMEM_SHARED((NPAD, DEG_W), jnp.float32),
        ],
    )
    def degk(dsts_hbm, out_hbm, didx, ones, zbuf, acc):
        cid = lax.axis_index("c")
        sid = lax.axis_index("s")
        wid = cid * 16 + sid
        base = wid * EROWS_W

        pltpu.sync_copy(dsts_hbm.at[pl.ds(base, EROWS_W)], didx)

        one = jnp.ones((16,), jnp.float32)
        kcols = DEG_W // 16

        def fill_ones(t, carry):
            ones[t // kcols, pl.ds((t % kcols) * 16, 16)] = one
            return carry

        lax.fori_loop(0, 128 * kcols, fill_ones, 0)
        _zero_fill(zbuf, 64, DEG_W)
        for t in range(ROWS_TILE // 64):
            pltpu.sync_copy(zbuf, acc.at[pl.ds(sid * ROWS_TILE + t * 64, 64)])
        plsc.subcore_barrier()

        def body(j, carry):
            pltpu.sync_copy(ones, acc.at[didx.at[j]], add=True)
            return carry

        lax.fori_loop(0, EROWS_W, body, 0)
        plsc.subcore_barrier()

        pltpu.sync_copy(
            acc.at[pl.ds(sid * ROWS_TILE, ROWS_TILE)],
            out_hbm.at[pl.ds(cid * NPAD + sid * ROWS_TILE, ROWS_TILE)],
        )

    return degk(dsts)


_R = 1024


def _row_mask(pid):
    rows = lax.broadcasted_iota(jnp.int32, (_R, 1), 0) + pid * _R
    return rows < N


def _tc_in(x, W_in, b_in, Wl0, Wr0, b0):
    def body(x_ref, wi_ref, bi_ref, wl_ref, wr_ref, b0_ref, inp_ref, m_ref, r_ref):
        t = jnp.dot(x_ref[...], wi_ref[...], preferred_element_type=jnp.float32)
        t = t + bi_ref[...][None, :]
        inp_ref[...] = t
        h = jnp.maximum(t, 0.0)
        m = jnp.dot(h, wl_ref[...], preferred_element_type=jnp.float32)
        m_ref[...] = jnp.where(_row_mask(pl.program_id(0)), m, 0.0)
        r_ref[...] = jnp.dot(h, wr_ref[...], preferred_element_type=jnp.float32) + b0_ref[...][None, :]

    grid = NPAD // _R
    blk = lambda d: pl.BlockSpec((_R, d), lambda i: (i, 0))
    wblk = lambda a, b: pl.BlockSpec((a, b), lambda i: (0, 0))
    vblk = lambda d: pl.BlockSpec((d,), lambda i: (0,))
    return pl.pallas_call(
        body,
        grid=(grid,),
        in_specs=[blk(D_IN), wblk(D_IN, D_H), vblk(D_H), wblk(D_H, D_H), wblk(D_H, D_H), vblk(D_H)],
        out_specs=[blk(D_H), blk(D_H), blk(D_H)],
        out_shape=[
            jax.ShapeDtypeStruct((NPAD, D_H), jnp.float32),
            jax.ShapeDtypeStruct((NPAD, D_H), jnp.float32),
            jax.ShapeDtypeStruct((NPAD, D_H), jnp.float32),
        ],
    )(x, W_in, b_in, Wl0, Wr0, b0)


def _tc_mid(S0, S1, d0, d1, r, inp, Wl, Wr, b, dout):
    def body(s0_ref, s1_ref, d0_ref, d1_ref, r_ref, inp_ref, wl_ref, wr_ref, b_ref,
             m_ref, rn_ref):
        deg = d0_ref[...][:, :1] + d1_ref[...][:, :1]
        scale = 1.0 / jnp.maximum(deg, 1.0)
        agg = (s0_ref[...] + s1_ref[...]) * scale
        h = jnp.maximum(agg + r_ref[...], 0.0) + 0.2 * inp_ref[...]
        m = jnp.dot(h, wl_ref[...], preferred_element_type=jnp.float32)
        m_ref[...] = jnp.where(_row_mask(pl.program_id(0)), m, 0.0)
        rn_ref[...] = jnp.dot(h, wr_ref[...], preferred_element_type=jnp.float32) + b_ref[...][None, :]

    grid = NPAD // _R
    blk = lambda d: pl.BlockSpec((_R, d), lambda i: (i, 0))
    wblk = lambda a, b: pl.BlockSpec((a, b), lambda i: (0, 0))
    vblk = lambda d: pl.BlockSpec((d,), lambda i: (0,))
    return pl.pallas_call(
        body,
        grid=(grid,),
        in_specs=[blk(D_H), blk(D_H), blk(DEG_W), blk(DEG_W), blk(D_H), blk(D_H),
                  wblk(D_H, dout), wblk(D_H, dout), vblk(dout)],
        out_specs=[blk(dout), blk(dout)],
        out_shape=[
            jax.ShapeDtypeStruct((NPAD, dout), jnp.float32),
            jax.ShapeDtypeStruct((NPAD, dout), jnp.float32),
        ],
    )(S0, S1, d0, d1, r, inp, Wl, Wr, b)


def _tc_final(S0, S1, d0, d1, r):
    def body(s0_ref, s1_ref, d0_ref, d1_ref, r_ref, o_ref):
        deg = d0_ref[...][:, :1] + d1_ref[...][:, :1]
        scale = 1.0 / jnp.maximum(deg, 1.0)
        logits = (s0_ref[...] + s1_ref[...]) * scale + r_ref[...]
        mx = jnp.max(logits, axis=-1, keepdims=True)
        sh = logits - mx
        lse = jnp.log(jnp.sum(jnp.exp(sh), axis=-1, keepdims=True))
        o_ref[...] = sh - lse

    grid = NPAD // _R
    blk = lambda d: pl.BlockSpec((_R, d), lambda i: (i, 0))
    return pl.pallas_call(
        body,
        grid=(grid,),
        in_specs=[blk(D_OUT), blk(D_OUT), blk(DEG_W), blk(DEG_W), blk(D_OUT)],
        out_specs=blk(D_OUT),
        out_shape=jax.ShapeDtypeStruct((NPAD, D_OUT), jnp.float32),
    )(S0, S1, d0, d1, r)


def kernel(x, edge_index, W_in, b_in, Wl0, Wr0, b0, Wl1, Wr1, b1, Wl2, Wr2, b2):
    src = edge_index[0]
    dst = edge_index[1]
    pad = jnp.full((EPAD - E,), N, jnp.int32)
    srcs = jnp.concatenate([src, pad]).reshape(EROWS, 128)
    dsts = jnp.concatenate([dst, pad]).reshape(EROWS, 128)
    x_pad = jnp.pad(x, ((0, NPAD - N), (0, 0)))

    degp = _sc_degree(dsts)
    d0, d1 = degp[:NPAD], degp[NPAD:]

    inp, m0, r0 = _tc_in(x_pad, W_in, b_in, Wl0, Wr0, b0)

    S = _sc_scatter_sum(m0, srcs, dsts, D_H)
    m1, r1 = _tc_mid(S[:NPAD], S[NPAD:], d0, d1, r0, inp, Wl1, Wr1, b1, D_H)

    S = _sc_scatter_sum(m1, srcs, dsts, D_H)
    Wl2p = jnp.pad(Wl2, ((0, 0), (0, D_H - D_OUT)))
    Wr2p = jnp.pad(Wr2, ((0, 0), (0, D_H - D_OUT)))
    b2p = jnp.pad(b2, (0, D_H - D_OUT))
    m2, r2 = _tc_mid(S[:NPAD], S[NPAD:], d0, d1, r1, inp, Wl2p, Wr2p, b2p, D_H)

    S = _sc_scatter_sum(m2, srcs, dsts, D_H)
    out = _tc_final(S[:NPAD, :D_OUT], S[NPAD:NPAD + NPAD, :D_OUT], d0, d1, r2[:, :D_OUT])
    return out[:N]


# speedup vs baseline: 3.4215x; 1.0720x over previous
import functools

import jax
import jax.numpy as jnp
from jax import lax
from jax.experimental import pallas as pl
from jax.experimental.pallas import tpu as pltpu
from jax.experimental.pallas import tpu_sc as plsc

N = 10000
E = 320000
D_IN = 128
D_H = 128
D_OUT = 64

NPAD = 10240
EPAD = 327680
EROWS = EPAD // 128
NW = 32
EROWS_W = EROWS // NW
ROWS_TILE = NPAD // 16
DEG_W = 128

_mesh = plsc.VectorSubcoreMesh(core_axis_name="c", subcore_axis_name="s")


def _zero_fill(ref, nrows, ncols):
    z = jnp.zeros((16,), jnp.float32)
    kcols = ncols // 16

    def body(t, carry):
        i = t // kcols
        k = t % kcols
        ref[i, pl.ds(k * 16, 16)] = z
        return carry

    lax.fori_loop(0, nrows * kcols, body, 0)


_CH = 16


def _sc_scatter_sum(m, srcs, dsts, zeros, dm):
    nslot = 2

    @functools.partial(
        pl.kernel,
        out_type=jax.ShapeDtypeStruct((2 * NPAD, dm), jnp.float32),
        mesh=_mesh,
        scratch_types=[
            pltpu.VMEM((_CH, 128), jnp.int32),
            pltpu.VMEM((_CH, 128), jnp.int32),
            pltpu.VMEM((nslot, 128, dm), jnp.float32),
            pltpu.VMEM_SHARED((NPAD, dm), jnp.float32),
            pltpu.SemaphoreType.DMA((nslot,)),
            pltpu.SemaphoreType.DMA((nslot,)),
        ],
    )
    def scat(m_hbm, srcs_hbm, dsts_hbm, zeros_hbm, out_hbm, sidx, didx, rows, acc, gsem, ssem):
        cid = lax.axis_index("c")
        sid = lax.axis_index("s")
        wid = cid * 16 + sid
        base = wid * EROWS_W

        pltpu.sync_copy(
            zeros_hbm.at[pl.ds(sid * ROWS_TILE, ROWS_TILE)],
            acc.at[pl.ds(sid * ROWS_TILE, ROWS_TILE)],
        )
        plsc.subcore_barrier()

        def chunk(ci, carry):
            pltpu.sync_copy(srcs_hbm.at[pl.ds(base + ci * _CH, _CH)], sidx)
            pltpu.sync_copy(dsts_hbm.at[pl.ds(base + ci * _CH, _CH)], didx)

            for s in range(nslot):
                pltpu.async_copy(m_hbm.at[sidx.at[s]], rows.at[s], gsem.at[s])

            def body(i, c2):
                for s in range(nslot):
                    j = i * nslot + s
                    pltpu.make_async_copy(m_hbm.at[sidx.at[j]], rows.at[s], gsem.at[s]).wait()
                    pltpu.async_copy(rows.at[s], acc.at[didx.at[j]], ssem.at[s], add=True)

                    @pl.when(j + nslot < _CH)
                    def _():
                        pltpu.make_async_copy(rows.at[s], acc.at[didx.at[j]], ssem.at[s]).wait()
                        pltpu.async_copy(m_hbm.at[sidx.at[j + nslot]], rows.at[s], gsem.at[s])

                return c2

            lax.fori_loop(0, _CH // nslot, body, 0)
            for s in range(nslot):
                pltpu.make_async_copy(rows.at[s], acc.at[didx.at[s]], ssem.at[s]).wait()
            return carry

        lax.fori_loop(0, EROWS_W // _CH, chunk, 0)
        plsc.subcore_barrier()

        pltpu.sync_copy(
            acc.at[pl.ds(sid * ROWS_TILE, ROWS_TILE)],
            out_hbm.at[pl.ds(cid * NPAD + sid * ROWS_TILE, ROWS_TILE)],
        )

    return scat(m, srcs, dsts, zeros)


def _sc_degree(dsts):
    @functools.partial(
        pl.kernel,
        out_type=jax.ShapeDtypeStruct((2 * NPAD, DEG_W), jnp.float32),
        mesh=_mesh,
        scratch_types=[
            pltpu.VMEM((EROWS_W, 128), jnp.int32),
            pltpu.VMEM((128, DEG_W), jnp.float32),
            pltpu.VMEM((64, DEG_W), jnp.float32),
            pltpu.VMEM_SHARED((NPAD, DEG_W), jnp.float32),
            pltpu.SemaphoreType.DMA,
        ],
    )
    def degk(dsts_hbm, out_hbm, didx, ones, zbuf, acc, sem):
        cid = lax.axis_index("c")
        sid = lax.axis_index("s")
        wid = cid * 16 + sid
        base = wid * EROWS_W

        pltpu.sync_copy(dsts_hbm.at[pl.ds(base, EROWS_W)], didx)

        one = jnp.ones((16,), jnp.float32)
        kcols = DEG_W // 16

        def fill_ones(t, carry):
            ones[t // kcols, pl.ds((t % kcols) * 16, 16)] = one
            return carry

        lax.fori_loop(0, 128 * kcols, fill_ones, 0)
        _zero_fill(zbuf, 64, DEG_W)
        for t in range(ROWS_TILE // 64):
            pltpu.sync_copy(zbuf, acc.at[pl.ds(sid * ROWS_TILE + t * 64, 64)])
        plsc.subcore_barrier()

        def body(j, carry):
            pltpu.sync_copy(ones, acc.at[didx.at[j]], add=True)
            return carry

        lax.fori_loop(0, EROWS_W, body, 0)
        plsc.subcore_barrier()

        pltpu.sync_copy(
            acc.at[pl.ds(sid * ROWS_TILE, ROWS_TILE)],
            out_hbm.at[pl.ds(cid * NPAD + sid * ROWS_TILE, ROWS_TILE)],
        )

    return degk(dsts)


_R = 1024


def _row_mask(pid):
    rows = lax.broadcasted_iota(jnp.int32, (_R, 1), 0) + pid * _R
    return rows < N


def _tc_in(x, W_in, b_in, Wl0, Wr0, b0):
    def body(x_ref, wi_ref, bi_ref, wl_ref, wr_ref, b0_ref, inp_ref, m_ref, r_ref):
        t = jnp.dot(x_ref[...], wi_ref[...], preferred_element_type=jnp.float32)
        t = t + bi_ref[...][None, :]
        inp_ref[...] = t
        h = jnp.maximum(t, 0.0)
        m = jnp.dot(h, wl_ref[...], preferred_element_type=jnp.float32)
        m_ref[...] = jnp.where(_row_mask(pl.program_id(0)), m, 0.0)
        r_ref[...] = jnp.dot(h, wr_ref[...], preferred_element_type=jnp.float32) + b0_ref[...][None, :]

    grid = NPAD // _R
    blk = lambda d: pl.BlockSpec((_R, d), lambda i: (i, 0))
    wblk = lambda a, b: pl.BlockSpec((a, b), lambda i: (0, 0))
    vblk = lambda d: pl.BlockSpec((d,), lambda i: (0,))
    return pl.pallas_call(
        body,
        grid=(grid,),
        in_specs=[blk(D_IN), wblk(D_IN, D_H), vblk(D_H), wblk(D_H, D_H), wblk(D_H, D_H), vblk(D_H)],
        out_specs=[blk(D_H), blk(D_H), blk(D_H)],
        out_shape=[
            jax.ShapeDtypeStruct((NPAD, D_H), jnp.float32),
            jax.ShapeDtypeStruct((NPAD, D_H), jnp.float32),
            jax.ShapeDtypeStruct((NPAD, D_H), jnp.float32),
        ],
    )(x, W_in, b_in, Wl0, Wr0, b0)


def _tc_mid(S0, S1, d0, d1, r, inp, Wl, Wr, b, dout):
    def body(s0_ref, s1_ref, d0_ref, d1_ref, r_ref, inp_ref, wl_ref, wr_ref, b_ref,
             m_ref, rn_ref):
        deg = d0_ref[...][:, :1] + d1_ref[...][:, :1]
        scale = 1.0 / jnp.maximum(deg, 1.0)
        agg = (s0_ref[...] + s1_ref[...]) * scale
        h = jnp.maximum(agg + r_ref[...], 0.0) + 0.2 * inp_ref[...]
        m = jnp.dot(h, wl_ref[...], preferred_element_type=jnp.float32)
        m_ref[...] = jnp.where(_row_mask(pl.program_id(0)), m, 0.0)
        rn_ref[...] = jnp.dot(h, wr_ref[...], preferred_element_type=jnp.float32) + b_ref[...][None, :]

    grid = NPAD // _R
    blk = lambda d: pl.BlockSpec((_R, d), lambda i: (i, 0))
    wblk = lambda a, b: pl.BlockSpec((a, b), lambda i: (0, 0))
    vblk = lambda d: pl.BlockSpec((d,), lambda i: (0,))
    return pl.pallas_call(
        body,
        grid=(grid,),
        in_specs=[blk(D_H), blk(D_H), blk(DEG_W), blk(DEG_W), blk(D_H), blk(D_H),
                  wblk(D_H, dout), wblk(D_H, dout), vblk(dout)],
        out_specs=[blk(dout), blk(dout)],
        out_shape=[
            jax.ShapeDtypeStruct((NPAD, dout), jnp.float32),
            jax.ShapeDtypeStruct((NPAD, dout), jnp.float32),
        ],
    )(S0, S1, d0, d1, r, inp, Wl, Wr, b)


def _tc_final(S0, S1, d0, d1, r):
    def body(s0_ref, s1_ref, d0_ref, d1_ref, r_ref, o_ref):
        deg = d0_ref[...][:, :1] + d1_ref[...][:, :1]
        scale = 1.0 / jnp.maximum(deg, 1.0)
        logits = (s0_ref[...] + s1_ref[...]) * scale + r_ref[...]
        mx = jnp.max(logits, axis=-1, keepdims=True)
        sh = logits - mx
        lse = jnp.log(jnp.sum(jnp.exp(sh), axis=-1, keepdims=True))
        o_ref[...] = sh - lse

    grid = NPAD // _R
    blk = lambda d: pl.BlockSpec((_R, d), lambda i: (i, 0))
    return pl.pallas_call(
        body,
        grid=(grid,),
        in_specs=[blk(D_OUT), blk(D_OUT), blk(DEG_W), blk(DEG_W), blk(D_OUT)],
        out_specs=blk(D_OUT),
        out_shape=jax.ShapeDtypeStruct((NPAD, D_OUT), jnp.float32),
    )(S0, S1, d0, d1, r)


def kernel(x, edge_index, W_in, b_in, Wl0, Wr0, b0, Wl1, Wr1, b1, Wl2, Wr2, b2):
    src = edge_index[0]
    dst = edge_index[1]
    pad = jnp.full((EPAD - E,), N, jnp.int32)
    srcs = jnp.concatenate([src, pad]).reshape(EROWS, 128)
    dsts = jnp.concatenate([dst, pad]).reshape(EROWS, 128)
    x_pad = jnp.pad(x, ((0, NPAD - N), (0, 0)))
    zeros = jnp.zeros((NPAD, D_H), jnp.float32)

    degp = _sc_degree(dsts)
    d0, d1 = degp[:NPAD], degp[NPAD:]

    inp, m0, r0 = _tc_in(x_pad, W_in, b_in, Wl0, Wr0, b0)

    S = _sc_scatter_sum(m0, srcs, dsts, zeros, D_H)
    m1, r1 = _tc_mid(S[:NPAD], S[NPAD:], d0, d1, r0, inp, Wl1, Wr1, b1, D_H)

    S = _sc_scatter_sum(m1, srcs, dsts, zeros, D_H)
    Wl2p = jnp.pad(Wl2, ((0, 0), (0, D_H - D_OUT)))
    Wr2p = jnp.pad(Wr2, ((0, 0), (0, D_H - D_OUT)))
    b2p = jnp.pad(b2, (0, D_H - D_OUT))
    m2, r2 = _tc_mid(S[:NPAD], S[NPAD:], d0, d1, r1, inp, Wl2p, Wr2p, b2p, D_H)

    S = _sc_scatter_sum(m2, srcs, dsts, zeros, D_H)
    out = _tc_final(S[:NPAD, :D_OUT], S[NPAD:NPAD + NPAD, :D_OUT], d0, d1, r2[:, :D_OUT])
    return out[:N]
